# baseline (speedup 1.0000x reference)
import jax
import jax.numpy as jnp
from jax.experimental import pallas as pl
from jax.experimental.pallas import tpu as pltpu

N_NODES = 10000
NFEAT = 128
EMBED = 32
NSTREAMS = 2
ROWS = 200
CHUNK = N_NODES // NSTREAMS
STEPS = CHUNK // ROWS


def _gcn_kernel(feat_ref, *refs):
    adj_refs = refs[:NSTREAMS]
    w_ref = refs[NSTREAMS]
    out_ref = refs[NSTREAMS + 1]
    support_ref = refs[NSTREAMS + 2]
    i = pl.program_id(0)

    @pl.when(i == 0)
    def _():
        support_ref[...] = jnp.dot(
            feat_ref[...], w_ref[...], preferred_element_type=jnp.float32
        )

    for s in range(NSTREAMS):
        out_ref[s] = jnp.dot(
            adj_refs[s][...], support_ref[...], preferred_element_type=jnp.float32
        )


@jax.jit
def kernel(features, adj, W):
    adj_specs = [
        pl.BlockSpec((ROWS, N_NODES), lambda i, s=s: (i + s * STEPS, 0))
        for s in range(NSTREAMS)
    ]
    out = pl.pallas_call(
        _gcn_kernel,
        grid=(STEPS,),
        in_specs=[
            pl.BlockSpec((N_NODES, NFEAT), lambda i: (0, 0)),
            *adj_specs,
            pl.BlockSpec((NFEAT, EMBED), lambda i: (0, 0)),
        ],
        out_specs=pl.BlockSpec((NSTREAMS, ROWS, EMBED), lambda i: (0, i, 0)),
        out_shape=jax.ShapeDtypeStruct((NSTREAMS, CHUNK, EMBED), jnp.float32),
        scratch_shapes=[pltpu.VMEM((N_NODES, EMBED), jnp.float32)],
        compiler_params=pltpu.CompilerParams(
            dimension_semantics=("arbitrary",),
        ),
    )(features, *([adj] * NSTREAMS), W)
    return out.reshape(N_NODES, EMBED)


# speedup vs baseline: 1.0010x; 1.0010x over previous
import jax
import jax.numpy as jnp
from jax.experimental import pallas as pl
from jax.experimental.pallas import tpu as pltpu

N_NODES = 10000
NFEAT = 128
EMBED = 32
ROWS = 400


def _band_kernel(support_ref, adj_ref, out_ref):
    out_ref[...] = jnp.dot(
        adj_ref[...], support_ref[...], preferred_element_type=jnp.float32
    )


@jax.jit
def kernel(features, adj, W):
    support = jnp.dot(features, W, preferred_element_type=jnp.float32)
    grid = (N_NODES // ROWS,)
    return pl.pallas_call(
        _band_kernel,
        grid=grid,
        in_specs=[
            pl.BlockSpec((N_NODES, EMBED), lambda i: (0, 0)),
            pl.BlockSpec((ROWS, N_NODES), lambda i: (i, 0)),
        ],
        out_specs=pl.BlockSpec((ROWS, EMBED), lambda i: (i, 0)),
        out_shape=jax.ShapeDtypeStruct((N_NODES, EMBED), jnp.float32),
        compiler_params=pltpu.CompilerParams(
            dimension_semantics=("arbitrary",),
        ),
    )(support, adj)
